# Initial kernel scaffold; baseline (speedup 1.0000x reference)
import jax
import jax.numpy as jnp
from jax.experimental import pallas as pl


def kernel(x, edge_index, batch, W1, b1, W2, b2, Wfc, bfc):
    raise NotImplementedError("write your pallas kernel here")


# speedup vs baseline: 13.2666x; 13.2666x over previous
import functools

import jax
import jax.numpy as jnp
from jax import lax
from jax.experimental import pallas as pl
from jax.experimental.pallas import tpu as pltpu
from jax.experimental.pallas import tpu_sc as plsc

NC = 2
NS = 16
LANES = 16
G_SEG = 64


def _mesh():
    return plsc.VectorSubcoreMesh(core_axis_name="c", subcore_axis_name="s")


def _row_split(N):
    RPB = 640
    RLAST = N - RPB * (NS - 1)
    CHR = 80
    assert RPB % CHR == 0 and RLAST % CHR == 0 and RLAST > 0
    return RPB, RLAST, CHR


def _make_deg_kernel(N, E):
    NW = NC * NS
    EPW = E // NW
    CH = 80
    NCHUNK = EPW // CH
    RPB, RLAST, CHR = _row_split(N)

    @functools.partial(
        pl.kernel,
        out_type=jax.ShapeDtypeStruct((NC, N, LANES), jnp.float32),
        mesh=_mesh(),
        scratch_types=[
            pltpu.VMEM((CH,), jnp.int32),
            pltpu.VMEM((CH, LANES), jnp.float32),
            pltpu.VMEM((CHR, LANES), jnp.float32),
            pltpu.VMEM_SHARED((N, LANES), jnp.float32),
            pltpu.SemaphoreType.DMA,
        ],
    )
    def deg_kernel(ei_hbm, out_hbm, idxd_v, ones_v, zbuf_v, acc_sh, sem):
        cid = lax.axis_index("c")
        sid = lax.axis_index("s")
        wid = cid * NS + sid
        base = wid * EPW
        rstart = sid * RPB

        ones16 = jnp.ones((LANES,), jnp.float32)
        zeros16 = jnp.zeros((LANES,), jnp.float32)

        @pl.loop(0, CH)
        def _(r):
            ones_v[r, :] = ones16

        @pl.loop(0, CHR)
        def _(r):
            zbuf_v[r, :] = zeros16

        nrows = jnp.where(sid == NS - 1, RLAST, RPB)

        @pl.loop(0, nrows, step=CHR)
        def _(r0):
            pltpu.sync_copy(zbuf_v, acc_sh.at[pl.ds(rstart + r0, CHR)])

        plsc.subcore_barrier()

        @pl.loop(0, NCHUNK)
        def _(c):
            off = base + c * CH
            pltpu.sync_copy(ei_hbm.at[pl.ds(E + off, CH)], idxd_v)
            pltpu.sync_copy(ones_v, acc_sh.at[idxd_v], add=True)

        plsc.subcore_barrier()

        @pl.loop(0, nrows, step=CHR)
        def _(r0):
            r = rstart + r0
            pltpu.sync_copy(acc_sh.at[pl.ds(r, CHR)],
                            out_hbm.at[cid, pl.ds(r, CHR)])

    return deg_kernel


def _make_edge_pass(N, E, Dh):
    NW = NC * NS
    EPW = E // NW
    CH = 80
    NCHUNK = EPW // CH
    RPB, RLAST, CHR = _row_split(N)

    @functools.partial(
        pl.kernel,
        out_type=jax.ShapeDtypeStruct((NC, N, Dh), jnp.float32),
        mesh=_mesh(),
        scratch_types=[
            pltpu.VMEM((CH,), jnp.int32),
            pltpu.VMEM((CH,), jnp.int32),
            pltpu.VMEM((CH, Dh), jnp.float32),
            pltpu.VMEM((CHR, Dh), jnp.float32),
            pltpu.VMEM_SHARED((N, Dh), jnp.float32),
            pltpu.SemaphoreType.DMA,
        ],
    )
    def edge_pass(u_hbm, ei_hbm, out_hbm,
                  idxs_v, idxd_v, rows_v, zbuf_v, acc_sh, sem):
        cid = lax.axis_index("c")
        sid = lax.axis_index("s")
        wid = cid * NS + sid
        base = wid * EPW
        rstart = sid * RPB

        zeros16 = jnp.zeros((LANES,), jnp.float32)

        @pl.loop(0, CHR)
        def _(r):
            @pl.loop(0, Dh, step=LANES)
            def _(j):
                zbuf_v[r, pl.ds(j, LANES)] = zeros16

        nrows = jnp.where(sid == NS - 1, RLAST, RPB)

        @pl.loop(0, nrows, step=CHR)
        def _(r0):
            pltpu.sync_copy(zbuf_v, acc_sh.at[pl.ds(rstart + r0, CHR)])

        plsc.subcore_barrier()

        @pl.loop(0, NCHUNK)
        def _(c):
            off = base + c * CH
            pltpu.sync_copy(ei_hbm.at[pl.ds(off, CH)], idxs_v)
            pltpu.sync_copy(ei_hbm.at[pl.ds(E + off, CH)], idxd_v)
            pltpu.async_copy(u_hbm.at[idxs_v], rows_v, sem).wait()
            pltpu.sync_copy(rows_v, acc_sh.at[idxd_v], add=True)

        plsc.subcore_barrier()

        @pl.loop(0, nrows, step=CHR)
        def _(r0):
            r = rstart + r0
            pltpu.sync_copy(acc_sh.at[pl.ds(r, CHR)],
                            out_hbm.at[cid, pl.ds(r, CHR)])

    return edge_pass


def _deg_dinv(degp_blk):
    deg = jnp.sum(degp_blk, axis=(0, 2)) * (1.0 / LANES) + 1.0
    return lax.rsqrt(deg)


def _u1_body(degp_ref, x_ref, w_ref, o_ref):
    dinv = _deg_dinv(degp_ref[...])
    y = jnp.dot(x_ref[...], w_ref[...],
                preferred_element_type=jnp.float32,
                precision=lax.Precision.HIGHEST)
    o_ref[...] = y * dinv[:, None]


def _u2_body(degp_ref, aggp_ref, u1_ref, b1_ref, w2_ref, o_ref):
    dinv = _deg_dinv(degp_ref[...])
    agg = aggp_ref[0] + aggp_ref[1] + u1_ref[...]
    h1 = jnp.maximum(agg * dinv[:, None] + b1_ref[...], 0.0)
    y = jnp.dot(h1, w2_ref[...],
                preferred_element_type=jnp.float32,
                precision=lax.Precision.HIGHEST)
    o_ref[...] = y * dinv[:, None]


def _make_final_body(NBLK, BR, Hdim, O):
    def final_body(degp_ref, aggp_ref, u2_ref, b2_ref, batch_ref,
                   wfc_ref, bfc_ref, o_ref, pooled_acc, cnt_acc):
        i = pl.program_id(0)

        @pl.when(i == 0)
        def _():
            pooled_acc[...] = jnp.zeros_like(pooled_acc)
            cnt_acc[...] = jnp.zeros_like(cnt_acc)

        dinv = _deg_dinv(degp_ref[...])
        agg = aggp_ref[0] + aggp_ref[1] + u2_ref[...]
        h2 = jnp.maximum(agg * dinv[:, None] + b2_ref[...], 0.0)

        b = batch_ref[0, 0, :]
        gid = lax.broadcasted_iota(jnp.int32, (G_SEG, BR), 0)
        oh = jnp.where(gid == b[None, :], 1.0, 0.0)
        pooled_acc[...] += jnp.dot(oh, h2,
                                   preferred_element_type=jnp.float32,
                                   precision=lax.Precision.HIGHEST)
        cnt_acc[...] += jnp.dot(oh, jnp.ones((BR, Hdim), jnp.float32),
                                preferred_element_type=jnp.float32,
                                precision=lax.Precision.HIGHEST)

        @pl.when(i == NBLK - 1)
        def _():
            pooled = pooled_acc[...] / jnp.maximum(cnt_acc[...], 1.0)
            logits = jnp.dot(pooled, wfc_ref[...],
                             preferred_element_type=jnp.float32,
                             precision=lax.Precision.HIGHEST) + bfc_ref[...]
            m = jnp.max(logits, axis=1, keepdims=True)
            z = logits - m
            o_ref[...] = z - jnp.log(jnp.sum(jnp.exp(z), axis=1,
                                             keepdims=True))

    return final_body


def kernel(x, edge_index, batch, W1, b1, W2, b2, Wfc, bfc):
    N, D = x.shape
    Hdim = W1.shape[1]
    E = edge_index.shape[1]
    O = Wfc.shape[1]
    BR = 1000
    NBLK = N // BR

    deg_kernel = _make_deg_kernel(N, E)
    edge_pass = _make_edge_pass(N, E, Hdim)

    ei_flat = edge_index.reshape(2 * E)
    degp = deg_kernel(ei_flat)

    u1 = pl.pallas_call(
        _u1_body,
        grid=(NBLK,),
        in_specs=[
            pl.BlockSpec((NC, BR, LANES), lambda i: (0, i, 0)),
            pl.BlockSpec((BR, D), lambda i: (i, 0)),
            pl.BlockSpec((D, Hdim), lambda i: (0, 0)),
        ],
        out_specs=pl.BlockSpec((BR, Hdim), lambda i: (i, 0)),
        out_shape=jax.ShapeDtypeStruct((N, Hdim), jnp.float32),
    )(degp, x, W1)

    agg1 = edge_pass(u1, ei_flat)

    u2 = pl.pallas_call(
        _u2_body,
        grid=(NBLK,),
        in_specs=[
            pl.BlockSpec((NC, BR, LANES), lambda i: (0, i, 0)),
            pl.BlockSpec((NC, BR, Hdim), lambda i: (0, i, 0)),
            pl.BlockSpec((BR, Hdim), lambda i: (i, 0)),
            pl.BlockSpec((1, Hdim), lambda i: (0, 0)),
            pl.BlockSpec((Hdim, Hdim), lambda i: (0, 0)),
        ],
        out_specs=pl.BlockSpec((BR, Hdim), lambda i: (i, 0)),
        out_shape=jax.ShapeDtypeStruct((N, Hdim), jnp.float32),
    )(degp, agg1, u1, b1.reshape(1, Hdim), W2)

    agg2 = edge_pass(u2, ei_flat)

    out = pl.pallas_call(
        _make_final_body(NBLK, BR, Hdim, O),
        grid=(NBLK,),
        in_specs=[
            pl.BlockSpec((NC, BR, LANES), lambda i: (0, i, 0)),
            pl.BlockSpec((NC, BR, Hdim), lambda i: (0, i, 0)),
            pl.BlockSpec((BR, Hdim), lambda i: (i, 0)),
            pl.BlockSpec((1, Hdim), lambda i: (0, 0)),
            pl.BlockSpec((1, 1, BR), lambda i: (i, 0, 0)),
            pl.BlockSpec((Hdim, O), lambda i: (0, 0)),
            pl.BlockSpec((1, O), lambda i: (0, 0)),
        ],
        out_specs=pl.BlockSpec((G_SEG, O), lambda i: (0, 0)),
        out_shape=jax.ShapeDtypeStruct((G_SEG, O), jnp.float32),
        scratch_shapes=[
            pltpu.VMEM((G_SEG, Hdim), jnp.float32),
            pltpu.VMEM((G_SEG, Hdim), jnp.float32),
        ],
    )(degp, agg2, u2, b2.reshape(1, Hdim),
      batch.reshape(NBLK, 1, BR), Wfc, bfc.reshape(1, O))

    return out


# speedup vs baseline: 34.5551x; 2.6047x over previous
import functools

import jax
import jax.numpy as jnp
from jax import lax
from jax.experimental import pallas as pl
from jax.experimental.pallas import tpu as pltpu
from jax.experimental.pallas import tpu_sc as plsc

NC = 2
NS = 16
LANES = 16
G_SEG = 64

CH = 80
NB = 4
IDXR = 2 * NB


def _mesh():
    return plsc.VectorSubcoreMesh(core_axis_name="c", subcore_axis_name="s")


def _row_split(N):
    RPB = 640
    RLAST = N - RPB * (NS - 1)
    CHR = 80
    assert RPB % CHR == 0 and RLAST % CHR == 0 and RLAST > 0
    return RPB, RLAST, CHR


def _make_deg_kernel(N, E):
    NW = NC * NS
    EPW = E // NW
    NCHUNK = EPW // CH
    MAIN = (NCHUNK // IDXR) * IDXR
    SD = 4
    RPB, RLAST, _ = _row_split(N)

    @functools.partial(
        pl.kernel,
        out_type=jax.ShapeDtypeStruct((NC, N, LANES), jnp.float32),
        mesh=_mesh(),
        scratch_types=(
            [pltpu.VMEM((CH, LANES), jnp.float32),
             pltpu.VMEM_SHARED((N, LANES), jnp.float32)]
            + [pltpu.VMEM((CH,), jnp.int32) for _ in range(IDXR)]
            + [pltpu.SemaphoreType.DMA for _ in range(IDXR)]
            + [pltpu.SemaphoreType.DMA for _ in range(IDXR)]
            + [pltpu.SemaphoreType.DMA]
        ),
    )
    def deg_kernel(ei_hbm, ones_hbm, z_hbm, out_hbm,
                   ones_v, acc_sh, *bufs):
        dstb = bufs[:IDXR]
        isems = bufs[IDXR:2 * IDXR]
        ssems = bufs[2 * IDXR:3 * IDXR]
        zsem = bufs[3 * IDXR]

        cid = lax.axis_index("c")
        sid = lax.axis_index("s")
        wid = cid * NS + sid
        base = wid * EPW
        rstart = sid * RPB

        def issue_idx(k, isl):
            pltpu.async_copy(ei_hbm.at[pl.ds(E + base + k * CH, CH)],
                             dstb[isl], isems[isl])

        def wait_idx(k, isl):
            pltpu.make_async_copy(ei_hbm.at[pl.ds(E + base + k * CH, CH)],
                                  dstb[isl], isems[isl]).wait()

        def fire_scatter(k, isl):
            pltpu.async_copy(ones_v, acc_sh.at[dstb[isl]], ssems[isl],
                             add=True)

        def wait_scatter(k, isl):
            pltpu.make_async_copy(ones_v, acc_sh.at[dstb[isl]],
                                  ssems[isl]).wait()

        pltpu.async_copy(ones_hbm, ones_v, zsem)

        @pl.when(sid < NS - 1)
        def _():
            pltpu.async_copy(z_hbm, acc_sh.at[pl.ds(rstart, RPB)], zsem)

        @pl.when(sid == NS - 1)
        def _():
            pltpu.async_copy(z_hbm.at[pl.ds(0, RLAST)],
                             acc_sh.at[pl.ds(rstart, RLAST)], zsem)

        for j in range(SD):
            issue_idx(j, j)

        pltpu.make_async_copy(ones_hbm, ones_v, zsem).wait()

        @pl.when(sid < NS - 1)
        def _():
            pltpu.make_async_copy(z_hbm, acc_sh.at[pl.ds(rstart, RPB)],
                                  zsem).wait()

        @pl.when(sid == NS - 1)
        def _():
            pltpu.make_async_copy(z_hbm.at[pl.ds(0, RLAST)],
                                  acc_sh.at[pl.ds(rstart, RLAST)],
                                  zsem).wait()

        plsc.subcore_barrier()

        def chunk_step(k, j, static):
            s = j % IDXR
            s4 = (j + SD) % IDXR
            wait_idx(k, s)
            fire_scatter(k, s)

            def lag():
                wait_scatter(k - SD, s4)

            def ahead():
                issue_idx(k + SD, s4)

            if static:
                if k >= SD:
                    lag()
                if k + SD < NCHUNK:
                    ahead()
            else:
                lag()

                @pl.when(k + SD < NCHUNK)
                def _():
                    ahead()

        for j in range(IDXR):
            chunk_step(j, j, True)

        @pl.loop(1, MAIN // IDXR)
        def _(c0):
            k0 = c0 * IDXR
            for j in range(IDXR):
                chunk_step(k0 + j, j, False)

        for j in range(NCHUNK - MAIN):
            chunk_step(MAIN + j, j, True)

        for k in range(NCHUNK - SD, NCHUNK):
            wait_scatter(k, k % IDXR)

        plsc.subcore_barrier()

        @pl.when(sid < NS - 1)
        def _():
            pltpu.sync_copy(acc_sh.at[pl.ds(rstart, RPB)],
                            out_hbm.at[cid, pl.ds(rstart, RPB)])

        @pl.when(sid == NS - 1)
        def _():
            pltpu.sync_copy(acc_sh.at[pl.ds(rstart, RLAST)],
                            out_hbm.at[cid, pl.ds(rstart, RLAST)])

    return deg_kernel


def _make_edge_pass(N, E, Dh):
    NW = NC * NS
    EPW = E // NW
    NCHUNK = EPW // CH
    MAIN = (NCHUNK // IDXR) * IDXR
    RPB, RLAST, _ = _row_split(N)

    @functools.partial(
        pl.kernel,
        out_type=jax.ShapeDtypeStruct((NC, N, Dh), jnp.float32),
        mesh=_mesh(),
        scratch_types=(
            [pltpu.VMEM_SHARED((N, Dh), jnp.float32)]
            + [pltpu.VMEM((CH,), jnp.int32) for _ in range(2 * IDXR)]
            + [pltpu.VMEM((CH, Dh), jnp.float32) for _ in range(NB)]
            + [pltpu.SemaphoreType.DMA for _ in range(NB)]
            + [pltpu.SemaphoreType.DMA for _ in range(IDXR)]
            + [pltpu.SemaphoreType.DMA]
        ),
    )
    def edge_pass(u_hbm, ei_hbm, z_hbm, out_hbm, acc_sh, *bufs):
        srcb = bufs[:IDXR]
        dstb = bufs[IDXR:2 * IDXR]
        bufs = bufs[2 * IDXR:]
        rows = bufs[:NB]
        gsems = bufs[NB:2 * NB]
        isems = bufs[2 * NB:2 * NB + IDXR]
        zsem = bufs[2 * NB + IDXR]

        cid = lax.axis_index("c")
        sid = lax.axis_index("s")
        wid = cid * NS + sid
        base = wid * EPW
        rstart = sid * RPB

        def issue_idx(k, isl):
            pltpu.async_copy(ei_hbm.at[pl.ds(base + k * CH, CH)],
                             srcb[isl], isems[isl])
            pltpu.async_copy(ei_hbm.at[pl.ds(E + base + k * CH, CH)],
                             dstb[isl], isems[isl])

        def wait_idx(k, isl):
            pltpu.make_async_copy(ei_hbm.at[pl.ds(base + k * CH, CH)],
                                  srcb[isl], isems[isl]).wait()
            pltpu.make_async_copy(ei_hbm.at[pl.ds(E + base + k * CH, CH)],
                                  dstb[isl], isems[isl]).wait()

        def issue_gather(k, gs, isl):
            pltpu.async_copy(u_hbm.at[srcb[isl]], rows[gs], gsems[gs])

        def wait_gather(k, gs, isl):
            pltpu.make_async_copy(u_hbm.at[srcb[isl]], rows[gs],
                                  gsems[gs]).wait()

        @pl.when(sid < NS - 1)
        def _():
            pltpu.async_copy(z_hbm, acc_sh.at[pl.ds(rstart, RPB)], zsem)

        @pl.when(sid == NS - 1)
        def _():
            pltpu.async_copy(z_hbm.at[pl.ds(0, RLAST)],
                             acc_sh.at[pl.ds(rstart, RLAST)], zsem)

        for j in range(IDXR):
            issue_idx(j, j)

        @pl.when(sid < NS - 1)
        def _():
            pltpu.make_async_copy(z_hbm, acc_sh.at[pl.ds(rstart, RPB)],
                                  zsem).wait()

        @pl.when(sid == NS - 1)
        def _():
            pltpu.make_async_copy(z_hbm.at[pl.ds(0, RLAST)],
                                  acc_sh.at[pl.ds(rstart, RLAST)],
                                  zsem).wait()

        plsc.subcore_barrier()

        for j in range(NB):
            wait_idx(j, j)
            issue_gather(j, j, j)

        def chunk_step(k, j):
            gs = j % NB
            isl = j % IDXR
            wait_gather(k, gs, isl)
            pltpu.sync_copy(rows[gs], acc_sh.at[dstb[isl]], add=True)
            return gs, isl

        @pl.loop(0, MAIN // IDXR)
        def _(c0):
            k0 = c0 * IDXR
            for j in range(IDXR):
                k = k0 + j
                gs, isl = chunk_step(k, j)

                @pl.when(k + IDXR < NCHUNK)
                def _():
                    issue_idx(k + IDXR, isl)

                isl_n = (j + NB) % IDXR
                wait_idx(k + NB, isl_n)
                issue_gather(k + NB, gs, isl_n)

        for j in range(NCHUNK - MAIN):
            k = MAIN + j
            gs, isl = chunk_step(k, j)
            if k + NB < NCHUNK:
                isl_n = (j + NB) % IDXR
                wait_idx(k + NB, isl_n)
                issue_gather(k + NB, gs, isl_n)

        plsc.subcore_barrier()

        @pl.when(sid < NS - 1)
        def _():
            pltpu.sync_copy(acc_sh.at[pl.ds(rstart, RPB)],
                            out_hbm.at[cid, pl.ds(rstart, RPB)])

        @pl.when(sid == NS - 1)
        def _():
            pltpu.sync_copy(acc_sh.at[pl.ds(rstart, RLAST)],
                            out_hbm.at[cid, pl.ds(rstart, RLAST)])

    return edge_pass


def _deg_dinv(degp_blk):
    deg = jnp.sum(degp_blk, axis=(0, 2)) * (1.0 / LANES) + 1.0
    return lax.rsqrt(deg)


def _u1_body(degp_ref, x_ref, w_ref, o_ref):
    dinv = _deg_dinv(degp_ref[...])
    y = jnp.dot(x_ref[...], w_ref[...],
                preferred_element_type=jnp.float32,
                precision=lax.Precision.HIGHEST)
    o_ref[...] = y * dinv[:, None]


def _u2_body(degp_ref, aggp_ref, u1_ref, b1_ref, w2_ref, o_ref):
    dinv = _deg_dinv(degp_ref[...])
    agg = aggp_ref[0] + aggp_ref[1] + u1_ref[...]
    h1 = jnp.maximum(agg * dinv[:, None] + b1_ref[...], 0.0)
    y = jnp.dot(h1, w2_ref[...],
                preferred_element_type=jnp.float32,
                precision=lax.Precision.HIGHEST)
    o_ref[...] = y * dinv[:, None]


def _make_final_body(NBLK, BR, Hdim, O):
    def final_body(degp_ref, aggp_ref, u2_ref, b2_ref, batch_ref,
                   wfc_ref, bfc_ref, o_ref, pooled_acc, cnt_acc):
        i = pl.program_id(0)

        @pl.when(i == 0)
        def _():
            pooled_acc[...] = jnp.zeros_like(pooled_acc)
            cnt_acc[...] = jnp.zeros_like(cnt_acc)

        dinv = _deg_dinv(degp_ref[...])
        agg = aggp_ref[0] + aggp_ref[1] + u2_ref[...]
        h2 = jnp.maximum(agg * dinv[:, None] + b2_ref[...], 0.0)

        b = batch_ref[0, 0, :]
        gid = lax.broadcasted_iota(jnp.int32, (G_SEG, BR), 0)
        oh = jnp.where(gid == b[None, :], 1.0, 0.0)
        pooled_acc[...] += jnp.dot(oh, h2,
                                   preferred_element_type=jnp.float32,
                                   precision=lax.Precision.HIGHEST)
        cnt_acc[...] += jnp.dot(oh, jnp.ones((BR, Hdim), jnp.float32),
                                preferred_element_type=jnp.float32,
                                precision=lax.Precision.HIGHEST)

        @pl.when(i == NBLK - 1)
        def _():
            pooled = pooled_acc[...] / jnp.maximum(cnt_acc[...], 1.0)
            logits = jnp.dot(pooled, wfc_ref[...],
                             preferred_element_type=jnp.float32,
                             precision=lax.Precision.HIGHEST) + bfc_ref[...]
            m = jnp.max(logits, axis=1, keepdims=True)
            z = logits - m
            o_ref[...] = z - jnp.log(jnp.sum(jnp.exp(z), axis=1,
                                             keepdims=True))

    return final_body


def kernel(x, edge_index, batch, W1, b1, W2, b2, Wfc, bfc):
    N, D = x.shape
    Hdim = W1.shape[1]
    E = edge_index.shape[1]
    O = Wfc.shape[1]
    BR = 1000
    NBLK = N // BR
    NW = NC * NS
    NCHUNK = E // NW // CH

    deg_kernel = _make_deg_kernel(N, E)
    edge_pass = _make_edge_pass(N, E, Hdim)

    ei_flat = edge_index.reshape(2 * E)
    RPB = _row_split(N)[0]
    zeros_rows = jnp.zeros((RPB, Hdim), jnp.float32)
    degp = deg_kernel(ei_flat,
                      jnp.ones((CH, LANES), jnp.float32),
                      jnp.zeros((RPB, LANES), jnp.float32))

    u1 = pl.pallas_call(
        _u1_body,
        grid=(NBLK,),
        in_specs=[
            pl.BlockSpec((NC, BR, LANES), lambda i: (0, i, 0)),
            pl.BlockSpec((BR, D), lambda i: (i, 0)),
            pl.BlockSpec((D, Hdim), lambda i: (0, 0)),
        ],
        out_specs=pl.BlockSpec((BR, Hdim), lambda i: (i, 0)),
        out_shape=jax.ShapeDtypeStruct((N, Hdim), jnp.float32),
    )(degp, x, W1)

    agg1 = edge_pass(u1, ei_flat, zeros_rows)

    u2 = pl.pallas_call(
        _u2_body,
        grid=(NBLK,),
        in_specs=[
            pl.BlockSpec((NC, BR, LANES), lambda i: (0, i, 0)),
            pl.BlockSpec((NC, BR, Hdim), lambda i: (0, i, 0)),
            pl.BlockSpec((BR, Hdim), lambda i: (i, 0)),
            pl.BlockSpec((1, Hdim), lambda i: (0, 0)),
            pl.BlockSpec((Hdim, Hdim), lambda i: (0, 0)),
        ],
        out_specs=pl.BlockSpec((BR, Hdim), lambda i: (i, 0)),
        out_shape=jax.ShapeDtypeStruct((N, Hdim), jnp.float32),
    )(degp, agg1, u1, b1.reshape(1, Hdim), W2)

    agg2 = edge_pass(u2, ei_flat, zeros_rows)

    out = pl.pallas_call(
        _make_final_body(NBLK, BR, Hdim, O),
        grid=(NBLK,),
        in_specs=[
            pl.BlockSpec((NC, BR, LANES), lambda i: (0, i, 0)),
            pl.BlockSpec((NC, BR, Hdim), lambda i: (0, i, 0)),
            pl.BlockSpec((BR, Hdim), lambda i: (i, 0)),
            pl.BlockSpec((1, Hdim), lambda i: (0, 0)),
            pl.BlockSpec((1, 1, BR), lambda i: (i, 0, 0)),
            pl.BlockSpec((Hdim, O), lambda i: (0, 0)),
            pl.BlockSpec((1, O), lambda i: (0, 0)),
        ],
        out_specs=pl.BlockSpec((G_SEG, O), lambda i: (0, 0)),
        out_shape=jax.ShapeDtypeStruct((G_SEG, O), jnp.float32),
        scratch_shapes=[
            pltpu.VMEM((G_SEG, Hdim), jnp.float32),
            pltpu.VMEM((G_SEG, Hdim), jnp.float32),
        ],
    )(degp, agg2, u2, b2.reshape(1, Hdim),
      batch.reshape(NBLK, 1, BR), Wfc, bfc.reshape(1, O))

    return out
